# baseline (speedup 1.0000x reference)
import functools

import jax
import jax.numpy as jnp
from jax import lax
from jax.experimental import pallas as pl
from jax.experimental.pallas import tpu as pltpu
from jax.experimental.pallas import tpu_sc as plsc

N = 100000
D_EMB = 32
D_CBO = 16
D_ENC = 80
D_OUT = D_EMB + D_CBO + D_ENC
V = 256

BLK = 256
NBLK_BIG = 13
NBLK_SMALL = 12
IDX_SPAN = NBLK_BIG * BLK
N_ALIGNED = 99968
EXTRA_COL = N_ALIGNED - BLK
TAIL = N - N_ALIGNED

_info = plsc.get_sparse_core_info()
NC = _info.num_cores
NS = _info.num_subcores
NW = NC * NS

_mesh = plsc.VectorSubcoreMesh(core_axis_name="c", subcore_axis_name="s")


@functools.partial(
    pl.kernel,
    mesh=_mesh,
    out_type=jax.ShapeDtypeStruct((D_OUT, N), jnp.float32),
    scratch_types=[
        pltpu.VMEM((IDX_SPAN,), jnp.int32),
        pltpu.VMEM((D_EMB, V), jnp.float32),
        pltpu.VMEM((D_OUT, BLK), jnp.float32),
        pltpu.VMEM((D_OUT, BLK), jnp.float32),
        pltpu.VMEM((D_OUT, BLK), jnp.float32),
        pltpu.SemaphoreType.DMA,
        pltpu.SemaphoreType.DMA,
        pltpu.SemaphoreType.DMA,
        pltpu.SemaphoreType.DMA,
        pltpu.SemaphoreType.DMA,
        pltpu.SemaphoreType.DMA,
        pltpu.SemaphoreType.DMA,
        pltpu.SemaphoreType.DMA,
        pltpu.SemaphoreType.DMA,
    ],
    compiler_params=pltpu.CompilerParams(needs_layout_passes=False),
)
def _embed(gid, cboT, encT, tableT, outT,
           idx_all, tab_v, blk0, blk1, blk2,
           cs0, cs1, cs2, es0, es1, es2, os0, os1, os2):
    wid = lax.axis_index("s") * NC + lax.axis_index("c")
    big = wid < 6
    last = wid == NW - 1
    base_w = pl.multiple_of(
        wid * (NBLK_SMALL * BLK) + jnp.minimum(wid, 6) * BLK, 128)
    nblk = jnp.where(big, NBLK_BIG, NBLK_SMALL)
    blks = (blk0, blk1, blk2)
    css = (cs0, cs1, cs2)
    ess = (es0, es1, es2)
    oss = (os0, os1, os2)

    def col_of(t):
        return pl.multiple_of(base_w + t * BLK, 128)

    def issue_in(t, p):
        col = col_of(t)
        pltpu.async_copy(cboT.at[:, pl.ds(col, BLK)],
                         blks[p].at[pl.ds(D_EMB, D_CBO), :], css[p])
        pltpu.async_copy(encT.at[:, pl.ds(col, BLK)],
                         blks[p].at[pl.ds(D_EMB + D_CBO, D_ENC), :], ess[p])

    def wait_in(t, p):
        col = col_of(t)
        pltpu.make_async_copy(cboT.at[:, pl.ds(col, BLK)],
                              blks[p].at[pl.ds(D_EMB, D_CBO), :],
                              css[p]).wait()
        pltpu.make_async_copy(encT.at[:, pl.ds(col, BLK)],
                              blks[p].at[pl.ds(D_EMB + D_CBO, D_ENC), :],
                              ess[p]).wait()

    def issue_out(t, p):
        pltpu.async_copy(blks[p], outT.at[:, pl.ds(col_of(t), BLK)], oss[p])

    def wait_out(t, p):
        pltpu.make_async_copy(blks[p], outT.at[:, pl.ds(col_of(t), BLK)],
                              oss[p]).wait()

    issue_in(0, 0)
    issue_in(1, 1)
    pltpu.sync_copy(tableT, tab_v)
    pltpu.sync_copy(gid.at[pl.ds(base_w, NBLK_SMALL * BLK)],
                    idx_all.at[pl.ds(0, NBLK_SMALL * BLK)])

    @pl.when(big)
    def _():
        pltpu.sync_copy(gid.at[pl.ds(base_w + NBLK_SMALL * BLK, BLK)],
                        idx_all.at[pl.ds(NBLK_SMALL * BLK, BLK)])

    @pl.when(last)
    def _():
        pltpu.sync_copy(gid.at[pl.ds(base_w + NBLK_SMALL * BLK, 128)],
                        idx_all.at[pl.ds(NBLK_SMALL * BLK, 128)])

    def emb_block(t, p):
        blk_v = blks[p]

        def group(g, _):
            off = g * 16
            idx16 = idx_all[pl.ds(t * BLK + off, 16)]
            for c in range(D_EMB):
                v = plsc.load_gather(
                    tab_v, [jnp.full((16,), c, jnp.int32), idx16])
                blk_v[c, pl.ds(off, 16)] = v
            return _

        lax.fori_loop(0, BLK // 16, group, None)

    def triple(u, _):
        for h in (0, 1, 2):
            t = u * 3 + h

            @pl.when(t < nblk)
            def _():
                emb_block(t, h)
                wait_in(t, h)
                issue_out(t, h)

                @pl.when(t >= 1)
                def _():
                    wait_out(t - 1, (h + 2) % 3)

                @pl.when(t + 2 < nblk)
                def _():
                    issue_in(t + 2, (h + 2) % 3)

        return _

    lax.fori_loop(0, (NBLK_BIG + 2) // 3, triple, None)

    @pl.when(big)
    def _():
        wait_out(NBLK_BIG - 1, (NBLK_BIG - 1) % 3)

    @pl.when(jnp.logical_not(big))
    def _():
        wait_out(NBLK_SMALL - 1, (NBLK_SMALL - 1) % 3)

    @pl.when(last)
    def _():
        c1 = pltpu.async_copy(cboT.at[:, pl.ds(EXTRA_COL, BLK)],
                              blk0.at[pl.ds(D_EMB, D_CBO), :], cs0)
        c2 = pltpu.async_copy(encT.at[:, pl.ds(EXTRA_COL, BLK)],
                              blk0.at[pl.ds(D_EMB + D_CBO, D_ENC), :], es0)

        def group(g, _):
            off = g * 16
            idx16 = idx_all[pl.ds(EXTRA_COL - base_w + off, 16)]
            for c in range(D_EMB):
                v = plsc.load_gather(
                    tab_v, [jnp.full((16,), c, jnp.int32), idx16])
                blk0[c, pl.ds(off, 16)] = v
            return _

        lax.fori_loop(0, BLK // 16, group, None)
        c1.wait()
        c2.wait()
        pltpu.sync_copy(blk0, outT.at[:, pl.ds(EXTRA_COL, BLK)])


def _tc_tail(gid_tail, cboT_tail, encT_tail, tableT, outT):
    def body(gid_ref, cbo_ref, enc_ref, tab_ref, alias_ref, out_ref):
        del alias_ref
        ids = gid_ref[0, :]
        iot = lax.broadcasted_iota(jnp.int32, (V, TAIL), 0)
        onehot = jnp.where(iot == ids[None, :], 1.0, 0.0)
        embT = jnp.dot(tab_ref[...], onehot,
                       preferred_element_type=jnp.float32,
                       precision=lax.Precision.HIGHEST)
        out_ref[pl.ds(0, D_EMB), pl.ds(0, TAIL)] = embT
        out_ref[pl.ds(D_EMB, D_CBO), pl.ds(0, TAIL)] = cbo_ref[...]
        out_ref[pl.ds(D_EMB + D_CBO, D_ENC), pl.ds(0, TAIL)] = enc_ref[...]

    return pl.pallas_call(
        body,
        grid=(1,),
        in_specs=[
            pl.BlockSpec((1, TAIL), lambda i: (0, 0)),
            pl.BlockSpec((D_CBO, TAIL), lambda i: (0, 0)),
            pl.BlockSpec((D_ENC, TAIL), lambda i: (0, 0)),
            pl.BlockSpec((D_EMB, V), lambda i: (0, 0)),
            pl.BlockSpec(memory_space=pl.ANY),
        ],
        out_specs=pl.BlockSpec((D_OUT, 128), lambda i: (0, N_ALIGNED // 128)),
        out_shape=jax.ShapeDtypeStruct((D_OUT, N), jnp.float32),
        input_output_aliases={4: 0},
    )(gid_tail, cboT_tail, encT_tail, tableT, outT)


def kernel(op_gid, cbo, enc, op_table):
    gid32 = op_gid.astype(jnp.int32)
    cboT = cbo.T
    encT = enc.T
    tableT = op_table.T
    outT = _embed(gid32, cboT, encT, tableT)
    outT = _tc_tail(gid32[N_ALIGNED:].reshape(1, TAIL),
                    cboT[:, N_ALIGNED:], encT[:, N_ALIGNED:], tableT, outT)
    return outT.T


# speedup vs baseline: 1.0246x; 1.0246x over previous
import functools

import jax
import jax.numpy as jnp
from jax import lax
from jax.experimental import pallas as pl
from jax.experimental.pallas import tpu as pltpu
from jax.experimental.pallas import tpu_sc as plsc

N = 100000
D_EMB = 32
D_CBO = 16
D_ENC = 80
D_OUT = D_EMB + D_CBO + D_ENC
V = 256

BLK = 128
SPAN = 3200
NBLK_MAIN = SPAN // BLK
LAST_MAIN = 768
NBLK_LAST = LAST_MAIN // BLK
N_ALIGNED = SPAN * 31 + LAST_MAIN
TAIL = N - N_ALIGNED

_info = plsc.get_sparse_core_info()
NC = _info.num_cores
NS = _info.num_subcores
NW = NC * NS

_mesh = plsc.VectorSubcoreMesh(core_axis_name="c", subcore_axis_name="s")


@functools.partial(
    pl.kernel,
    mesh=_mesh,
    out_type=jax.ShapeDtypeStruct((D_OUT, N), jnp.float32),
    scratch_types=[
        pltpu.VMEM((SPAN,), jnp.int32),
        pltpu.VMEM((D_EMB, V), jnp.float32),
        pltpu.VMEM((D_OUT, BLK), jnp.float32),
        pltpu.VMEM((D_OUT, BLK), jnp.float32),
        pltpu.VMEM((D_OUT, BLK), jnp.float32),
        pltpu.VMEM((D_OUT, BLK), jnp.float32),
        pltpu.SemaphoreType.DMA,
        pltpu.SemaphoreType.DMA,
        pltpu.SemaphoreType.DMA,
        pltpu.SemaphoreType.DMA,
        pltpu.SemaphoreType.DMA,
        pltpu.SemaphoreType.DMA,
        pltpu.SemaphoreType.DMA,
        pltpu.SemaphoreType.DMA,
        pltpu.SemaphoreType.DMA,
        pltpu.SemaphoreType.DMA,
        pltpu.SemaphoreType.DMA,
        pltpu.SemaphoreType.DMA,
    ],
    compiler_params=pltpu.CompilerParams(needs_layout_passes=False),
)
def _embed(gid, cboT, encT, tableT, outT,
           idx_all, tab_v, blk0, blk1, blk2, blk3,
           cs0, cs1, cs2, cs3, es0, es1, es2, es3, os0, os1, os2, os3):
    wid = lax.axis_index("s") * NC + lax.axis_index("c")
    base_w = pl.multiple_of(wid * SPAN, 128)
    last = wid == NW - 1
    nblk = jnp.where(last, NBLK_LAST, NBLK_MAIN)
    blks = (blk0, blk1, blk2, blk3)
    css = (cs0, cs1, cs2, cs3)
    ess = (es0, es1, es2, es3)
    oss = (os0, os1, os2, os3)

    def col_of(t):
        return pl.multiple_of(base_w + t * BLK, 128)

    def issue_in(t, p):
        col = col_of(t)
        pltpu.async_copy(cboT.at[:, pl.ds(col, BLK)],
                         blks[p].at[pl.ds(D_EMB, D_CBO), :], css[p])
        pltpu.async_copy(encT.at[:, pl.ds(col, BLK)],
                         blks[p].at[pl.ds(D_EMB + D_CBO, D_ENC), :], ess[p])

    def wait_in(t, p):
        col = col_of(t)
        pltpu.make_async_copy(cboT.at[:, pl.ds(col, BLK)],
                              blks[p].at[pl.ds(D_EMB, D_CBO), :],
                              css[p]).wait()
        pltpu.make_async_copy(encT.at[:, pl.ds(col, BLK)],
                              blks[p].at[pl.ds(D_EMB + D_CBO, D_ENC), :],
                              ess[p]).wait()

    def issue_out(t, p):
        pltpu.async_copy(blks[p], outT.at[:, pl.ds(col_of(t), BLK)], oss[p])

    def wait_out(t, p):
        pltpu.make_async_copy(blks[p], outT.at[:, pl.ds(col_of(t), BLK)],
                              oss[p]).wait()

    issue_in(0, 0)
    issue_in(1, 1)
    pltpu.sync_copy(tableT, tab_v)
    pltpu.sync_copy(gid.at[pl.ds(base_w, LAST_MAIN)],
                    idx_all.at[pl.ds(0, LAST_MAIN)])

    @pl.when(jnp.logical_not(last))
    def _():
        pltpu.sync_copy(gid.at[pl.ds(base_w + LAST_MAIN, SPAN - LAST_MAIN)],
                        idx_all.at[pl.ds(LAST_MAIN, SPAN - LAST_MAIN)])

    def emb_block(t, p):
        blk_v = blks[p]

        def group(g, _):
            off = g * 16
            idx16 = idx_all[pl.ds(t * BLK + off, 16)]
            for c in range(D_EMB):
                v = plsc.load_gather(
                    tab_v, [jnp.full((16,), c, jnp.int32), idx16])
                blk_v[c, pl.ds(off, 16)] = v
            return _

        lax.fori_loop(0, BLK // 16, group, None)

    def quad(u, _):
        for h in (0, 1, 2, 3):
            t = u * 4 + h

            @pl.when(t < nblk)
            def _():
                emb_block(t, h)
                wait_in(t, h)
                issue_out(t, h)

                @pl.when(t >= 2)
                def _():
                    wait_out(t - 2, (h + 2) % 4)

                @pl.when(t + 2 < nblk)
                def _():
                    issue_in(t + 2, (h + 2) % 4)

        return _

    lax.fori_loop(0, (NBLK_MAIN + 3) // 4, quad, None)

    @pl.when(jnp.logical_not(last))
    def _():
        wait_out(NBLK_MAIN - 2, (NBLK_MAIN - 2) % 4)
        wait_out(NBLK_MAIN - 1, (NBLK_MAIN - 1) % 4)

    @pl.when(last)
    def _():
        wait_out(NBLK_LAST - 2, (NBLK_LAST - 2) % 4)
        wait_out(NBLK_LAST - 1, (NBLK_LAST - 1) % 4)


def _tc_tail(gid_tail, cboT_tail, encT_tail, tableT, outT):
    def body(gid_ref, cbo_ref, enc_ref, tab_ref, alias_ref, out_ref):
        del alias_ref
        ids = gid_ref[0, :]
        iot = lax.broadcasted_iota(jnp.int32, (V, TAIL), 0)
        onehot = jnp.where(iot == ids[None, :], 1.0, 0.0)
        embT = jnp.dot(tab_ref[...], onehot,
                       preferred_element_type=jnp.float32,
                       precision=lax.Precision.HIGHEST)
        out_ref[pl.ds(0, D_EMB), pl.ds(0, TAIL)] = embT
        out_ref[pl.ds(D_EMB, D_CBO), pl.ds(0, TAIL)] = cbo_ref[...]
        out_ref[pl.ds(D_EMB + D_CBO, D_ENC), pl.ds(0, TAIL)] = enc_ref[...]

    return pl.pallas_call(
        body,
        grid=(1,),
        in_specs=[
            pl.BlockSpec((1, TAIL), lambda i: (0, 0)),
            pl.BlockSpec((D_CBO, TAIL), lambda i: (0, 0)),
            pl.BlockSpec((D_ENC, TAIL), lambda i: (0, 0)),
            pl.BlockSpec((D_EMB, V), lambda i: (0, 0)),
            pl.BlockSpec(memory_space=pl.ANY),
        ],
        out_specs=pl.BlockSpec((D_OUT, 128), lambda i: (0, N_ALIGNED // 128)),
        out_shape=jax.ShapeDtypeStruct((D_OUT, N), jnp.float32),
        input_output_aliases={4: 0},
    )(gid_tail, cboT_tail, encT_tail, tableT, outT)


def kernel(op_gid, cbo, enc, op_table):
    gid32 = op_gid.astype(jnp.int32)
    cboT = cbo.T
    encT = enc.T
    tableT = op_table.T
    outT = _embed(gid32, cboT, encT, tableT)
    outT = _tc_tail(gid32[N_ALIGNED:].reshape(1, TAIL),
                    cboT[:, N_ALIGNED:], encT[:, N_ALIGNED:], tableT, outT)
    return outT.T


# speedup vs baseline: 1.0314x; 1.0067x over previous
import functools

import jax
import jax.numpy as jnp
from jax import lax
from jax.experimental import pallas as pl
from jax.experimental.pallas import tpu as pltpu
from jax.experimental.pallas import tpu_sc as plsc

N = 100000
D_EMB = 32
D_CBO = 16
D_ENC = 80
D_OUT = D_EMB + D_CBO + D_ENC
V = 256

BLK = 128
SPAN = 3200
NBLK_MAIN = SPAN // BLK
LAST_MAIN = 768
NBLK_LAST = LAST_MAIN // BLK
N_ALIGNED = SPAN * 31 + LAST_MAIN
TAIL = N - N_ALIGNED

_info = plsc.get_sparse_core_info()
NC = _info.num_cores
NS = _info.num_subcores
NW = NC * NS

_mesh = plsc.VectorSubcoreMesh(core_axis_name="c", subcore_axis_name="s")


@functools.partial(
    pl.kernel,
    mesh=_mesh,
    out_type=jax.ShapeDtypeStruct((D_OUT, N), jnp.float32),
    scratch_types=[
        pltpu.VMEM((SPAN,), jnp.int32),
        pltpu.VMEM((D_EMB, V), jnp.float32),
        pltpu.VMEM((D_OUT, BLK), jnp.float32),
        pltpu.VMEM((D_OUT, BLK), jnp.float32),
        pltpu.VMEM((D_OUT, BLK), jnp.float32),
        pltpu.SemaphoreType.DMA,
        pltpu.SemaphoreType.DMA,
        pltpu.SemaphoreType.DMA,
        pltpu.SemaphoreType.DMA,
        pltpu.SemaphoreType.DMA,
        pltpu.SemaphoreType.DMA,
        pltpu.SemaphoreType.DMA,
        pltpu.SemaphoreType.DMA,
        pltpu.SemaphoreType.DMA,
    ],
    compiler_params=pltpu.CompilerParams(needs_layout_passes=False),
)
def _embed(gid, cboT, encT, tableT, outT,
           idx_all, tab_v, blk0, blk1, blk2,
           cs0, cs1, cs2, es0, es1, es2, os0, os1, os2):
    wid = lax.axis_index("s") * NC + lax.axis_index("c")
    base_w = pl.multiple_of(wid * SPAN, 128)
    last = wid == NW - 1
    nblk = jnp.where(last, NBLK_LAST, NBLK_MAIN)
    blks = (blk0, blk1, blk2)
    css = (cs0, cs1, cs2)
    ess = (es0, es1, es2)
    oss = (os0, os1, os2)

    def col_of(t):
        return pl.multiple_of(base_w + t * BLK, 128)

    def issue_in(t, p):
        col = col_of(t)
        pltpu.async_copy(cboT.at[:, pl.ds(col, BLK)],
                         blks[p].at[pl.ds(D_EMB, D_CBO), :], css[p])
        pltpu.async_copy(encT.at[:, pl.ds(col, BLK)],
                         blks[p].at[pl.ds(D_EMB + D_CBO, D_ENC), :], ess[p])

    def wait_in(t, p):
        col = col_of(t)
        pltpu.make_async_copy(cboT.at[:, pl.ds(col, BLK)],
                              blks[p].at[pl.ds(D_EMB, D_CBO), :],
                              css[p]).wait()
        pltpu.make_async_copy(encT.at[:, pl.ds(col, BLK)],
                              blks[p].at[pl.ds(D_EMB + D_CBO, D_ENC), :],
                              ess[p]).wait()

    def issue_out(t, p):
        pltpu.async_copy(blks[p], outT.at[:, pl.ds(col_of(t), BLK)], oss[p])

    def wait_out(t, p):
        pltpu.make_async_copy(blks[p], outT.at[:, pl.ds(col_of(t), BLK)],
                              oss[p]).wait()

    issue_in(0, 0)
    issue_in(1, 1)
    pltpu.sync_copy(tableT, tab_v)
    pltpu.sync_copy(gid.at[pl.ds(base_w, LAST_MAIN)],
                    idx_all.at[pl.ds(0, LAST_MAIN)])

    @pl.when(jnp.logical_not(last))
    def _():
        pltpu.sync_copy(gid.at[pl.ds(base_w + LAST_MAIN, SPAN - LAST_MAIN)],
                        idx_all.at[pl.ds(LAST_MAIN, SPAN - LAST_MAIN)])

    def emb_block(t, p):
        blk_v = blks[p]

        def group(g, _):
            off = g * 16
            idx16 = idx_all[pl.ds(t * BLK + off, 16)]
            for c in range(D_EMB):
                v = plsc.load_gather(
                    tab_v, [jnp.full((16,), c, jnp.int32), idx16])
                blk_v[c, pl.ds(off, 16)] = v
            return _

        lax.fori_loop(0, BLK // 16, group, None)

    def triple(u, _):
        for h in (0, 1, 2):
            t = u * 3 + h

            @pl.when(t < nblk)
            def _():
                emb_block(t, h)
                wait_in(t, h)
                issue_out(t, h)

                @pl.when(t >= 1)
                def _():
                    wait_out(t - 1, (h + 2) % 3)

                @pl.when(t + 2 < nblk)
                def _():
                    issue_in(t + 2, (h + 2) % 3)

        return _

    lax.fori_loop(0, (NBLK_MAIN + 2) // 3, triple, None)

    @pl.when(jnp.logical_not(last))
    def _():
        wait_out(NBLK_MAIN - 1, (NBLK_MAIN - 1) % 3)

    @pl.when(last)
    def _():
        wait_out(NBLK_LAST - 1, (NBLK_LAST - 1) % 3)


def _tc_tail(gid_tail, cboT_tail, encT_tail, tableT, outT):
    def body(gid_ref, cbo_ref, enc_ref, tab_ref, alias_ref, out_ref):
        del alias_ref
        ids = gid_ref[0, :]
        iot = lax.broadcasted_iota(jnp.int32, (V, TAIL), 0)
        onehot = jnp.where(iot == ids[None, :], 1.0, 0.0)
        embT = jnp.dot(tab_ref[...], onehot,
                       preferred_element_type=jnp.float32,
                       precision=lax.Precision.HIGHEST)
        out_ref[pl.ds(0, D_EMB), pl.ds(0, TAIL)] = embT
        out_ref[pl.ds(D_EMB, D_CBO), pl.ds(0, TAIL)] = cbo_ref[...]
        out_ref[pl.ds(D_EMB + D_CBO, D_ENC), pl.ds(0, TAIL)] = enc_ref[...]

    return pl.pallas_call(
        body,
        grid=(1,),
        in_specs=[
            pl.BlockSpec((1, TAIL), lambda i: (0, 0)),
            pl.BlockSpec((D_CBO, TAIL), lambda i: (0, 0)),
            pl.BlockSpec((D_ENC, TAIL), lambda i: (0, 0)),
            pl.BlockSpec((D_EMB, V), lambda i: (0, 0)),
            pl.BlockSpec(memory_space=pl.ANY),
        ],
        out_specs=pl.BlockSpec((D_OUT, 128), lambda i: (0, N_ALIGNED // 128)),
        out_shape=jax.ShapeDtypeStruct((D_OUT, N), jnp.float32),
        input_output_aliases={4: 0},
    )(gid_tail, cboT_tail, encT_tail, tableT, outT)


def kernel(op_gid, cbo, enc, op_table):
    gid32 = op_gid.astype(jnp.int32)
    cboT = cbo.T
    encT = enc.T
    tableT = op_table.T
    outT = _embed(gid32, cboT, encT, tableT)
    outT = _tc_tail(gid32[N_ALIGNED:].reshape(1, TAIL),
                    cboT[:, N_ALIGNED:], encT[:, N_ALIGNED:], tableT, outT)
    return outT.T
